# Initial kernel scaffold; baseline (speedup 1.0000x reference)
import jax
import jax.numpy as jnp
from jax.experimental import pallas as pl


def kernel(feats, W1, b1, W2, b2, W_fc, b_fc, batch_ids):
    raise NotImplementedError("write your pallas kernel here")


# speedup vs baseline: 4.5842x; 4.5842x over previous
import jax
import jax.numpy as jnp
from jax import lax
from jax.experimental import pallas as pl
from jax.experimental.pallas import tpu as pltpu

B = 16
L = 4096
N = 32768
D_IN = 4
D_MID = 64
D_LOC = 16
D_OUT = 256

TB = 2048
K = L // TB
NPAD = N + L
NTA = NPAD // TB
GRID = NTA + B * K


def _body(feats_ref, ids_ref, W1_ref, b1_ref, W2_ref, b2c_ref, Wfc_ref,
          bfc_ref, out_ref, mfeat_ref, hbuf, pooled, cnt, offs):
    i = pl.program_id(0)

    @pl.when(i == 0)
    def _():
        ids = ids_ref[...]
        for b in range(B):
            cnt[b] = jnp.sum((ids == b).astype(jnp.int32))
            offs[b] = jnp.sum((ids < b).astype(jnp.int32))

    @pl.when(i < NTA)
    def _():
        f = feats_ref[...]
        h = jnp.maximum(
            jnp.dot(f, W1_ref[...], preferred_element_type=jnp.float32)
            + b1_ref[...], 0.0)
        hbuf[pl.ds(i * TB, TB), :] = h

    @pl.when(i >= NTA)
    def _():
        j = i - NTA
        b = j // K
        k = j % K
        cb = jnp.minimum(cnt[b], L)
        start = offs[b] + k * TB
        v = cb - k * TB
        hch = hbuf[pl.ds(start, TB), :]
        row_iota = lax.broadcasted_iota(jnp.int32, (TB, 1), 0)
        mfeat_ref[0] = jnp.where(row_iota < v, hch, 0.0)

        xT = lax.dot_general(W2_ref[...], hch, (((0,), (1,)), ((), ())),
                             preferred_element_type=jnp.float32) + b2c_ref[...]
        lane_iota = lax.broadcasted_iota(jnp.int32, (1, TB), 1)
        xTm = jnp.where(lane_iota < v, xT, xT[:, 0:1])
        cols = []
        for jj in range(D_LOC):
            prod = xTm * xTm[jj:jj + 1, :]
            cols.append(jnp.max(prod, axis=1, keepdims=True))
        tile = jnp.concatenate(cols, axis=1)
        flat = jnp.concatenate(
            [tile[ii:ii + 1, :] for ii in range(D_LOC)], axis=1)
        flat = jnp.where(v > 0, flat, jnp.full_like(flat, -1e30))

        @pl.when(k == 0)
        def _():
            pooled[pl.ds(b, 1), :] = flat

        @pl.when(k > 0)
        def _():
            pooled[pl.ds(b, 1), :] = jnp.maximum(pooled[pl.ds(b, 1), :], flat)

    @pl.when(i == GRID - 1)
    def _():
        P = pooled[...]
        pe = jnp.sign(P) * jnp.sqrt(jnp.abs(P) + 1e-8)
        nrm = jnp.sqrt(jnp.sum(pe * pe, axis=1, keepdims=True))
        flatn = pe / (nrm + 1e-12)
        out_ref[...] = jnp.dot(flatn, Wfc_ref[...],
                               preferred_element_type=jnp.float32) + bfc_ref[...]


def kernel(feats, W1, b1, W2, b2, W_fc, b_fc, batch_ids):
    feats_pad = jnp.concatenate(
        [feats, jnp.zeros((NPAD - N, D_IN), jnp.float32)], axis=0)
    ids2d = batch_ids.astype(jnp.int32).reshape(B, N // B)
    b1r = b1.reshape(1, D_MID)
    b2c = b2.reshape(D_LOC, 1)
    bfcr = b_fc.reshape(1, D_OUT)

    out, mfeat = pl.pallas_call(
        _body,
        grid=(GRID,),
        in_specs=[
            pl.BlockSpec((TB, D_IN), lambda i: (jnp.minimum(i, NTA - 1), 0)),
            pl.BlockSpec((B, N // B), lambda i: (0, 0)),
            pl.BlockSpec((D_IN, D_MID), lambda i: (0, 0)),
            pl.BlockSpec((1, D_MID), lambda i: (0, 0)),
            pl.BlockSpec((D_MID, D_LOC), lambda i: (0, 0)),
            pl.BlockSpec((D_LOC, 1), lambda i: (0, 0)),
            pl.BlockSpec((D_LOC * D_LOC, D_OUT), lambda i: (0, 0)),
            pl.BlockSpec((1, D_OUT), lambda i: (0, 0)),
        ],
        out_specs=[
            pl.BlockSpec((B, D_OUT), lambda i: (0, 0)),
            pl.BlockSpec(
                (1, TB, D_MID),
                lambda i: (jnp.maximum(i - NTA, 0) // K,
                           jnp.maximum(i - NTA, 0) % K, 0)),
        ],
        out_shape=[
            jax.ShapeDtypeStruct((B, D_OUT), jnp.float32),
            jax.ShapeDtypeStruct((B, L, D_MID), jnp.float32),
        ],
        scratch_shapes=[
            pltpu.VMEM((NPAD, D_MID), jnp.float32),
            pltpu.VMEM((B, D_LOC * D_LOC), jnp.float32),
            pltpu.SMEM((B,), jnp.int32),
            pltpu.SMEM((B,), jnp.int32),
        ],
        compiler_params=pltpu.CompilerParams(
            vmem_limit_bytes=100 * 1024 * 1024),
    )(feats_pad, ids2d, W1, b1r, W2, b2c, W_fc, bfcr)
    return out, mfeat


# speedup vs baseline: 5.3176x; 1.1600x over previous
import jax
import jax.numpy as jnp
from jax import lax
from jax.experimental import pallas as pl
from jax.experimental.pallas import tpu as pltpu

B = 16
L = 4096
N = 32768
D_IN = 4
D_MID = 64
D_LOC = 16
D_OUT = 256

TB = 4096
K = L // TB
NPAD = N + L
NTA = NPAD // TB
GRID = NTA + B * K


def _body(feats_ref, ids_ref, W1_ref, b1_ref, W2_ref, b2c_ref, Wfc_ref,
          bfc_ref, out_ref, mfeat_ref, hbuf, pooled, cnt, offs):
    i = pl.program_id(0)

    @pl.when(i == 0)
    def _():
        ids = ids_ref[...]
        for b in range(B):
            cnt[b] = jnp.sum((ids == b).astype(jnp.int32))
            offs[b] = jnp.sum((ids < b).astype(jnp.int32))

    @pl.when(i < NTA)
    def _():
        f = feats_ref[...]
        h = jnp.maximum(
            jnp.dot(f, W1_ref[...], preferred_element_type=jnp.float32)
            + b1_ref[...], 0.0)
        hbuf[pl.ds(i * TB, TB), :] = h

    @pl.when(i >= NTA)
    def _():
        j = i - NTA
        b = j // K
        k = j % K
        cb = jnp.minimum(cnt[b], L)
        start = offs[b] + k * TB
        v = cb - k * TB
        hch = hbuf[pl.ds(start, TB), :]
        row_iota = lax.broadcasted_iota(jnp.int32, (TB, 1), 0)
        mfeat_ref[0] = jnp.where(row_iota < v, hch, 0.0)

        xT = lax.dot_general(W2_ref[...], hch, (((0,), (1,)), ((), ())),
                             preferred_element_type=jnp.float32) + b2c_ref[...]
        lane_iota = lax.broadcasted_iota(jnp.int32, (1, TB), 1)
        xTm = jnp.where(lane_iota < v, xT, xT[:, 0:1])
        cols = []
        for jj in range(D_LOC):
            prod = xTm * xTm[jj:jj + 1, :]
            cols.append(jnp.max(prod, axis=1, keepdims=True))
        tile = jnp.concatenate(cols, axis=1)
        flat = jnp.concatenate(
            [tile[ii:ii + 1, :] for ii in range(D_LOC)], axis=1)
        flat = jnp.where(v > 0, flat, jnp.full_like(flat, -1e30))

        @pl.when(k == 0)
        def _():
            pooled[pl.ds(b, 1), :] = flat

        @pl.when(k > 0)
        def _():
            pooled[pl.ds(b, 1), :] = jnp.maximum(pooled[pl.ds(b, 1), :], flat)

    @pl.when(i == GRID - 1)
    def _():
        P = pooled[...]
        pe = jnp.sign(P) * jnp.sqrt(jnp.abs(P) + 1e-8)
        nrm = jnp.sqrt(jnp.sum(pe * pe, axis=1, keepdims=True))
        flatn = pe / (nrm + 1e-12)
        out_ref[...] = jnp.dot(flatn, Wfc_ref[...],
                               preferred_element_type=jnp.float32) + bfc_ref[...]


def kernel(feats, W1, b1, W2, b2, W_fc, b_fc, batch_ids):
    feats_pad = jnp.concatenate(
        [feats, jnp.zeros((NPAD - N, D_IN), jnp.float32)], axis=0)
    ids2d = batch_ids.astype(jnp.int32).reshape(B, N // B)
    b1r = b1.reshape(1, D_MID)
    b2c = b2.reshape(D_LOC, 1)
    bfcr = b_fc.reshape(1, D_OUT)

    out, mfeat = pl.pallas_call(
        _body,
        grid=(GRID,),
        in_specs=[
            pl.BlockSpec((TB, D_IN), lambda i: (jnp.minimum(i, NTA - 1), 0)),
            pl.BlockSpec((B, N // B), lambda i: (0, 0)),
            pl.BlockSpec((D_IN, D_MID), lambda i: (0, 0)),
            pl.BlockSpec((1, D_MID), lambda i: (0, 0)),
            pl.BlockSpec((D_MID, D_LOC), lambda i: (0, 0)),
            pl.BlockSpec((D_LOC, 1), lambda i: (0, 0)),
            pl.BlockSpec((D_LOC * D_LOC, D_OUT), lambda i: (0, 0)),
            pl.BlockSpec((1, D_OUT), lambda i: (0, 0)),
        ],
        out_specs=[
            pl.BlockSpec((B, D_OUT), lambda i: (0, 0)),
            pl.BlockSpec(
                (1, TB, D_MID),
                lambda i: (jnp.maximum(i - NTA, 0) // K,
                           jnp.maximum(i - NTA, 0) % K, 0)),
        ],
        out_shape=[
            jax.ShapeDtypeStruct((B, D_OUT), jnp.float32),
            jax.ShapeDtypeStruct((B, L, D_MID), jnp.float32),
        ],
        scratch_shapes=[
            pltpu.VMEM((NPAD, D_MID), jnp.float32),
            pltpu.VMEM((B, D_LOC * D_LOC), jnp.float32),
            pltpu.SMEM((B,), jnp.int32),
            pltpu.SMEM((B,), jnp.int32),
        ],
        compiler_params=pltpu.CompilerParams(
            vmem_limit_bytes=100 * 1024 * 1024),
    )(feats_pad, ids2d, W1, b1r, W2, b2c, W_fc, bfcr)
    return out, mfeat
